# baseline (speedup 1.0000x reference)
import functools

import jax
import jax.numpy as jnp
from jax import lax
from jax.experimental import pallas as pl
from jax.experimental.pallas import tpu as pltpu
from jax.experimental.pallas import tpu_sc as plsc

_N = 10000
_E = 320000
_H = 128
_B = 64
_HOPS = 5

_NC = 2
_NS = 16
_NW = _NC * _NS
_CH = 32
_CHUNKS_PER_TILE = 320
_E_PAD = _NW * _CHUNKS_PER_TILE * _CH
_RING = 5
_PHASES = 4
_PCHUNKS = _CHUNKS_PER_TILE // _PHASES
_N_PAD = 10240
_ROWS_PER_TILE = _N_PAD // _NS


def _gelu(x):
    return 0.5 * x * (1.0 + lax.erf(x * 0.7071067811865476))


def _ffnn(x, w1, b1, w2, b2):
    return _gelu(jnp.dot(x, w1) + b1) @ w2 + b2


def _sc_agg_body(h_hbm, src_hbm, dst_hbm, out_hbm,
                 agg_sp, src_v, dst_v, r0, r1, r2, r3, r4,
                 g0, g1, g2, g3, g4, t0, t1, t2, t3, t4):
    c = lax.axis_index("c")
    s = lax.axis_index("s")
    wid = c * _NS + s
    rows = (r0, r1, r2, r3, r4)
    gsem = (g0, g1, g2, g3, g4)
    ssem = (t0, t1, t2, t3, t4)

    def _zfill(i, carry):
        for j in range(_H // 16):
            r0[i, pl.ds(j * 16, 16)] = jnp.zeros((16,), jnp.float32)
        return carry
    lax.fori_loop(0, _CH, _zfill, 0)
    for k in range(_ROWS_PER_TILE // _CH):
        pltpu.sync_copy(r0, agg_sp.at[pl.ds(s * _ROWS_PER_TILE + k * _CH, _CH)])

    plsc.subcore_barrier()

    def _gather(k, b):
        pltpu.async_copy(h_hbm.at[src_v.at[k]], rows[b], gsem[b])

    def _gather_wait(k, b):
        pltpu.make_async_copy(h_hbm.at[src_v.at[k]], rows[b], gsem[b]).wait()

    def _scatter(k, b):
        pltpu.async_copy(rows[b], agg_sp.at[dst_v.at[k]], ssem[b], add=True)

    def _scatter_wait(k, b):
        pltpu.make_async_copy(rows[b], agg_sp.at[dst_v.at[k]], ssem[b]).wait()

    cbase = wid * _CHUNKS_PER_TILE
    for p in range(_PHASES):
        pltpu.sync_copy(src_hbm.at[pl.ds(cbase + p * _PCHUNKS, _PCHUNKS)], src_v)
        pltpu.sync_copy(dst_hbm.at[pl.ds(cbase + p * _PCHUNKS, _PCHUNKS)], dst_v)

        for b in range(_RING - 1):
            _gather(b, b)

        for b in range(_RING):
            _gather_wait(b, b)
            _scatter(b, b)
            if b >= 1:
                _scatter_wait(b - 1, b - 1)
            _gather(b + _RING - 1, (b + _RING - 1) % _RING)

        def _outer(k0, carry):
            for b in range(_RING):
                k = k0 * _RING + b
                _gather_wait(k, b)
                _scatter(k, b)
                _scatter_wait(k - 1, (b + _RING - 1) % _RING)
                _gather(k + _RING - 1, (b + _RING - 1) % _RING)
            return carry
        lax.fori_loop(1, _PCHUNKS // _RING - 1, _outer, 0)

        kl = _PCHUNKS - _RING
        for b in range(_RING):
            k = kl + b
            _gather_wait(k, b)
            _scatter(k, b)
            if b == 0:
                _scatter_wait(k - 1, _RING - 1)
                _gather(_PCHUNKS - 1, _RING - 1)
        for b in range(_RING):
            _scatter_wait(kl + b, b)

    plsc.subcore_barrier()
    pltpu.sync_copy(agg_sp.at[pl.ds(s * _ROWS_PER_TILE, _ROWS_PER_TILE)],
                    out_hbm.at[c].at[pl.ds(s * _ROWS_PER_TILE, _ROWS_PER_TILE)])


@functools.cache
def _sc_agg_kernel():
    return pl.kernel(
        _sc_agg_body,
        out_type=jax.ShapeDtypeStruct((_NC, _N_PAD, _H), jnp.float32),
        mesh=plsc.VectorSubcoreMesh(core_axis_name="c", subcore_axis_name="s",
                                    num_cores=_NC),
        scratch_types=[
            pltpu.VMEM_SHARED((_N_PAD, _H), jnp.float32),
            pltpu.VMEM((_PCHUNKS, _CH), jnp.int32),
            pltpu.VMEM((_PCHUNKS, _CH), jnp.int32),
            pltpu.VMEM((_CH, _H), jnp.float32),
            pltpu.VMEM((_CH, _H), jnp.float32),
            pltpu.VMEM((_CH, _H), jnp.float32),
            pltpu.VMEM((_CH, _H), jnp.float32),
            pltpu.VMEM((_CH, _H), jnp.float32),
            pltpu.SemaphoreType.DMA,
            pltpu.SemaphoreType.DMA,
            pltpu.SemaphoreType.DMA,
            pltpu.SemaphoreType.DMA,
            pltpu.SemaphoreType.DMA,
            pltpu.SemaphoreType.DMA,
            pltpu.SemaphoreType.DMA,
            pltpu.SemaphoreType.DMA,
            pltpu.SemaphoreType.DMA,
            pltpu.SemaphoreType.DMA,
        ],
    )


def _sc_agg(h, src2d, dst2d):
    return _sc_agg_kernel()(h, src2d, dst2d)


def _pre_body(x_ref, w1_ref, b1_ref, w2_ref, b2_ref, o_ref):
    o_ref[...] = _ffnn(x_ref[...], w1_ref[...], b1_ref[...],
                       w2_ref[...], b2_ref[...])


def _pre_call(x, p):
    return pl.pallas_call(
        _pre_body,
        out_shape=jax.ShapeDtypeStruct((_N, _H), jnp.float32),
    )(x, p["W1"], p["b1"].reshape(1, -1), p["W2"], p["b2"].reshape(1, -1))


def _vn_body(h_ref, brow_ref, vn_ref,
             uw1_ref, ub1_ref, uw2_ref, ub2_ref,
             pw1_ref, pb1_ref, pw2_ref, pb2_ref, o_ref):
    h = h_ref[...]
    oh_t = (lax.broadcasted_iota(jnp.int32, (_B, _N), 0)
            == brow_ref[...]).astype(jnp.float32)
    cnt = jnp.sum(oh_t, axis=1, keepdims=True)
    pool = jnp.dot(oh_t, h) / jnp.maximum(cnt, 1.0)
    vn = vn_ref[...] + _ffnn(pool, uw1_ref[...], ub1_ref[...],
                             uw2_ref[...], ub2_ref[...])
    o_ref[...] = _ffnn(vn, pw1_ref[...], pb1_ref[...],
                       pw2_ref[...], pb2_ref[...])


def _vn_call(h, brow, vn, upd, prop):
    return pl.pallas_call(
        _vn_body,
        out_shape=jax.ShapeDtypeStruct((_B, _H), jnp.float32),
    )(h, brow, vn,
      upd["W1"], upd["b1"].reshape(1, -1), upd["W2"], upd["b2"].reshape(1, -1),
      prop["W1"], prop["b1"].reshape(1, -1), prop["W2"], prop["b2"].reshape(1, -1))


def _upd_body(h_ref, parts_ref, outvn_ref, bcol_ref,
              gw1_ref, gb1_ref, gw2_ref, gb2_ref,
              fw1_ref, fb1_ref, fw2_ref, fb2_ref,
              bng_ref, bnb_ref, o_ref):
    h = h_ref[...]
    agg = parts_ref[0, :_N, :] + parts_ref[1, :_N, :]
    h1 = _ffnn(h + agg, gw1_ref[...], gb1_ref[...],
               gw2_ref[...], gb2_ref[...]) + h

    m = jnp.mean(h1, axis=0, keepdims=True)
    v = jnp.mean(h1 * h1, axis=0, keepdims=True) - m * m
    h1 = (h1 - m) / jnp.sqrt(v + 1e-5) * bng_ref[...] + bnb_ref[...]

    oh_n = (lax.broadcasted_iota(jnp.int32, (_N, _B), 1)
            == bcol_ref[...]).astype(jnp.float32)
    gath = jnp.dot(oh_n, outvn_ref[...])

    o_ref[...] = _ffnn(gath + h1, fw1_ref[...], fb1_ref[...],
                       fw2_ref[...], fb2_ref[...]) + h1


def _upd_call(h, parts, outvn, bcol, gin, ffnn, bng, bnb):
    return pl.pallas_call(
        _upd_body,
        out_shape=jax.ShapeDtypeStruct((_N, _H), jnp.float32),
    )(h, parts, outvn, bcol,
      gin["W1"], gin["b1"].reshape(1, -1), gin["W2"], gin["b2"].reshape(1, -1),
      ffnn["W1"], ffnn["b1"].reshape(1, -1), ffnn["W2"], ffnn["b2"].reshape(1, -1),
      bng.reshape(1, -1), bnb.reshape(1, -1))


def _post_body(h_ref, brow_ref, w1_ref, b1_ref, w2_ref, b2_ref, o_ref):
    h = h_ref[...]
    oh_t = (lax.broadcasted_iota(jnp.int32, (_B, _N), 0)
            == brow_ref[...]).astype(jnp.float32)
    cnt = jnp.sum(oh_t, axis=1, keepdims=True)
    pool = jnp.dot(oh_t, h) / jnp.maximum(cnt, 1.0)
    o_ref[...] = _ffnn(pool, w1_ref[...], b1_ref[...], w2_ref[...], b2_ref[...])


def _post_call(h, brow, p):
    return pl.pallas_call(
        _post_body,
        out_shape=jax.ShapeDtypeStruct((_B, _H), jnp.float32),
    )(h, brow, p["W1"], p["b1"].reshape(1, -1), p["W2"], p["b2"].reshape(1, -1))


def kernel(x, edge_index, batch, params):
    src = edge_index[0].astype(jnp.int32)
    dst = edge_index[1].astype(jnp.int32)
    pad = _E_PAD - _E
    ar = jnp.arange(pad, dtype=jnp.int32)
    src2d = jnp.concatenate([src, ar % _N]).reshape(-1, _CH)
    dst2d = jnp.concatenate([dst, _N + ar % (_N_PAD - _N)]).reshape(-1, _CH)
    brow = batch.reshape(1, _N).astype(jnp.int32)
    bcol = batch.reshape(_N, 1).astype(jnp.int32)

    h = _pre_call(x, params["pre"])
    for i in range(_HOPS):
        parts = _sc_agg(h, src2d, dst2d)
        outvn = _vn_call(h, brow, params["vn"],
                         params["upd"][i], params["prop"][i])
        h = _upd_call(h, parts, outvn, bcol,
                      params["gin"][i], params["ffnn"][i],
                      params["bn_g"][i], params["bn_b"][i])
    return _post_call(h, brow, params["post"])


# speedup vs baseline: 1.0077x; 1.0077x over previous
import functools

import jax
import jax.numpy as jnp
from jax import lax
from jax.experimental import pallas as pl
from jax.experimental.pallas import tpu as pltpu
from jax.experimental.pallas import tpu_sc as plsc

_N = 10000
_E = 320000
_H = 128
_B = 64
_HOPS = 5

_NC = 2
_NS = 16
_NW = _NC * _NS
_CH = 64
_CHUNKS_PER_TILE = 160
_E_PAD = _NW * _CHUNKS_PER_TILE * _CH
_RING = 4
_PHASES = 4
_PCHUNKS = _CHUNKS_PER_TILE // _PHASES
_N_PAD = 10240
_ROWS_PER_TILE = _N_PAD // _NS


def _gelu(x):
    return 0.5 * x * (1.0 + lax.erf(x * 0.7071067811865476))


def _ffnn(x, w1, b1, w2, b2):
    return _gelu(jnp.dot(x, w1) + b1) @ w2 + b2


def _sc_agg_body(h_hbm, src_hbm, dst_hbm, out_hbm,
                 agg_sp, src_v, dst_v, r0, r1, r2, r3,
                 g0, g1, g2, g3, t0, t1, t2, t3):
    c = lax.axis_index("c")
    s = lax.axis_index("s")
    wid = c * _NS + s
    rows = (r0, r1, r2, r3)
    gsem = (g0, g1, g2, g3)
    ssem = (t0, t1, t2, t3)

    def _zfill(i, carry):
        for j in range(_H // 16):
            r0[i, pl.ds(j * 16, 16)] = jnp.zeros((16,), jnp.float32)
        return carry
    lax.fori_loop(0, _CH, _zfill, 0)
    for k in range(_ROWS_PER_TILE // _CH):
        pltpu.sync_copy(r0, agg_sp.at[pl.ds(s * _ROWS_PER_TILE + k * _CH, _CH)])

    plsc.subcore_barrier()

    def _gather(k, b):
        pltpu.async_copy(h_hbm.at[src_v.at[k]], rows[b], gsem[b])

    def _gather_wait(k, b):
        pltpu.make_async_copy(h_hbm.at[src_v.at[k]], rows[b], gsem[b]).wait()

    def _scatter(k, b):
        pltpu.async_copy(rows[b], agg_sp.at[dst_v.at[k]], ssem[b], add=True)

    def _scatter_wait(k, b):
        pltpu.make_async_copy(rows[b], agg_sp.at[dst_v.at[k]], ssem[b]).wait()

    cbase = wid * _CHUNKS_PER_TILE
    for p in range(_PHASES):
        pltpu.sync_copy(src_hbm.at[pl.ds(cbase + p * _PCHUNKS, _PCHUNKS)], src_v)
        pltpu.sync_copy(dst_hbm.at[pl.ds(cbase + p * _PCHUNKS, _PCHUNKS)], dst_v)

        for b in range(_RING - 1):
            _gather(b, b)

        for b in range(_RING):
            _gather_wait(b, b)
            _scatter(b, b)
            if b >= 1:
                _scatter_wait(b - 1, b - 1)
            _gather(b + _RING - 1, (b + _RING - 1) % _RING)

        def _outer(k0, carry):
            for b in range(_RING):
                k = k0 * _RING + b
                _gather_wait(k, b)
                _scatter(k, b)
                _scatter_wait(k - 1, (b + _RING - 1) % _RING)
                _gather(k + _RING - 1, (b + _RING - 1) % _RING)
            return carry
        lax.fori_loop(1, _PCHUNKS // _RING - 1, _outer, 0)

        kl = _PCHUNKS - _RING
        for b in range(_RING):
            k = kl + b
            _gather_wait(k, b)
            _scatter(k, b)
            if b == 0:
                _scatter_wait(k - 1, _RING - 1)
                _gather(_PCHUNKS - 1, _RING - 1)
        for b in range(_RING):
            _scatter_wait(kl + b, b)

    plsc.subcore_barrier()
    pltpu.sync_copy(agg_sp.at[pl.ds(s * _ROWS_PER_TILE, _ROWS_PER_TILE)],
                    out_hbm.at[c].at[pl.ds(s * _ROWS_PER_TILE, _ROWS_PER_TILE)])


@functools.cache
def _sc_agg_kernel():
    return pl.kernel(
        _sc_agg_body,
        out_type=jax.ShapeDtypeStruct((_NC, _N_PAD, _H), jnp.float32),
        mesh=plsc.VectorSubcoreMesh(core_axis_name="c", subcore_axis_name="s",
                                    num_cores=_NC),
        scratch_types=[
            pltpu.VMEM_SHARED((_N_PAD, _H), jnp.float32),
            pltpu.VMEM((_PCHUNKS, _CH), jnp.int32),
            pltpu.VMEM((_PCHUNKS, _CH), jnp.int32),
            pltpu.VMEM((_CH, _H), jnp.float32),
            pltpu.VMEM((_CH, _H), jnp.float32),
            pltpu.VMEM((_CH, _H), jnp.float32),
            pltpu.VMEM((_CH, _H), jnp.float32),
            pltpu.SemaphoreType.DMA,
            pltpu.SemaphoreType.DMA,
            pltpu.SemaphoreType.DMA,
            pltpu.SemaphoreType.DMA,
            pltpu.SemaphoreType.DMA,
            pltpu.SemaphoreType.DMA,
            pltpu.SemaphoreType.DMA,
            pltpu.SemaphoreType.DMA,
        ],
    )


def _sc_agg(h, src2d, dst2d):
    return _sc_agg_kernel()(h, src2d, dst2d)


def _pre_body(x_ref, w1_ref, b1_ref, w2_ref, b2_ref, o_ref):
    o_ref[...] = _ffnn(x_ref[...], w1_ref[...], b1_ref[...],
                       w2_ref[...], b2_ref[...])


def _pre_call(x, p):
    return pl.pallas_call(
        _pre_body,
        out_shape=jax.ShapeDtypeStruct((_N, _H), jnp.float32),
    )(x, p["W1"], p["b1"].reshape(1, -1), p["W2"], p["b2"].reshape(1, -1))


def _vn_body(h_ref, brow_ref, vn_ref,
             uw1_ref, ub1_ref, uw2_ref, ub2_ref,
             pw1_ref, pb1_ref, pw2_ref, pb2_ref, o_ref):
    h = h_ref[...]
    oh_t = (lax.broadcasted_iota(jnp.int32, (_B, _N), 0)
            == brow_ref[...]).astype(jnp.float32)
    cnt = jnp.sum(oh_t, axis=1, keepdims=True)
    pool = jnp.dot(oh_t, h) / jnp.maximum(cnt, 1.0)
    vn = vn_ref[...] + _ffnn(pool, uw1_ref[...], ub1_ref[...],
                             uw2_ref[...], ub2_ref[...])
    o_ref[...] = _ffnn(vn, pw1_ref[...], pb1_ref[...],
                       pw2_ref[...], pb2_ref[...])


def _vn_call(h, brow, vn, upd, prop):
    return pl.pallas_call(
        _vn_body,
        out_shape=jax.ShapeDtypeStruct((_B, _H), jnp.float32),
    )(h, brow, vn,
      upd["W1"], upd["b1"].reshape(1, -1), upd["W2"], upd["b2"].reshape(1, -1),
      prop["W1"], prop["b1"].reshape(1, -1), prop["W2"], prop["b2"].reshape(1, -1))


def _upd_body(h_ref, parts_ref, outvn_ref, bcol_ref,
              gw1_ref, gb1_ref, gw2_ref, gb2_ref,
              fw1_ref, fb1_ref, fw2_ref, fb2_ref,
              bng_ref, bnb_ref, o_ref):
    h = h_ref[...]
    agg = parts_ref[0, :_N, :] + parts_ref[1, :_N, :]
    h1 = _ffnn(h + agg, gw1_ref[...], gb1_ref[...],
               gw2_ref[...], gb2_ref[...]) + h

    m = jnp.mean(h1, axis=0, keepdims=True)
    v = jnp.mean(h1 * h1, axis=0, keepdims=True) - m * m
    h1 = (h1 - m) / jnp.sqrt(v + 1e-5) * bng_ref[...] + bnb_ref[...]

    oh_n = (lax.broadcasted_iota(jnp.int32, (_N, _B), 1)
            == bcol_ref[...]).astype(jnp.float32)
    gath = jnp.dot(oh_n, outvn_ref[...])

    o_ref[...] = _ffnn(gath + h1, fw1_ref[...], fb1_ref[...],
                       fw2_ref[...], fb2_ref[...]) + h1


def _upd_call(h, parts, outvn, bcol, gin, ffnn, bng, bnb):
    return pl.pallas_call(
        _upd_body,
        out_shape=jax.ShapeDtypeStruct((_N, _H), jnp.float32),
    )(h, parts, outvn, bcol,
      gin["W1"], gin["b1"].reshape(1, -1), gin["W2"], gin["b2"].reshape(1, -1),
      ffnn["W1"], ffnn["b1"].reshape(1, -1), ffnn["W2"], ffnn["b2"].reshape(1, -1),
      bng.reshape(1, -1), bnb.reshape(1, -1))


def _post_body(h_ref, brow_ref, w1_ref, b1_ref, w2_ref, b2_ref, o_ref):
    h = h_ref[...]
    oh_t = (lax.broadcasted_iota(jnp.int32, (_B, _N), 0)
            == brow_ref[...]).astype(jnp.float32)
    cnt = jnp.sum(oh_t, axis=1, keepdims=True)
    pool = jnp.dot(oh_t, h) / jnp.maximum(cnt, 1.0)
    o_ref[...] = _ffnn(pool, w1_ref[...], b1_ref[...], w2_ref[...], b2_ref[...])


def _post_call(h, brow, p):
    return pl.pallas_call(
        _post_body,
        out_shape=jax.ShapeDtypeStruct((_B, _H), jnp.float32),
    )(h, brow, p["W1"], p["b1"].reshape(1, -1), p["W2"], p["b2"].reshape(1, -1))


def kernel(x, edge_index, batch, params):
    src = edge_index[0].astype(jnp.int32)
    dst = edge_index[1].astype(jnp.int32)
    pad = _E_PAD - _E
    ar = jnp.arange(pad, dtype=jnp.int32)
    src2d = jnp.concatenate([src, ar % _N]).reshape(-1, _CH)
    dst2d = jnp.concatenate([dst, _N + ar % (_N_PAD - _N)]).reshape(-1, _CH)
    brow = batch.reshape(1, _N).astype(jnp.int32)
    bcol = batch.reshape(_N, 1).astype(jnp.int32)

    h = _pre_call(x, params["pre"])
    for i in range(_HOPS):
        parts = _sc_agg(h, src2d, dst2d)
        outvn = _vn_call(h, brow, params["vn"],
                         params["upd"][i], params["prop"][i])
        h = _upd_call(h, parts, outvn, bcol,
                      params["gin"][i], params["ffnn"][i],
                      params["bn_g"][i], params["bn_b"][i])
    return _post_call(h, brow, params["post"])
